# baseline (speedup 1.0000x reference)
import functools

import jax
import jax.numpy as jnp
from jax import lax
from jax.experimental import pallas as pl
from jax.experimental.pallas import tpu as pltpu
from jax.experimental.pallas import tpu_sc as plsc

VOCAB = 100000
EMBED = 200
DPAD = 256
BATCH = 1024
CTX = 50

NC = 2
NS = 16
NW = NC * NS

B_PER_W = BATCH // NW
G = 80
NCHUNK = (B_PER_W * CTX) // G
ROWS_TOTAL = BATCH * CTX

PR = 4000


def _pad_body(t_ref, o_ref):
    o_ref[...] = jnp.concatenate(
        [t_ref[...], jnp.zeros((PR, DPAD - EMBED), jnp.float32)], axis=1)


def _pad_table(emb_table):
    return pl.pallas_call(
        _pad_body,
        grid=(VOCAB // PR,),
        in_specs=[pl.BlockSpec((PR, EMBED), lambda i: (i, 0))],
        out_specs=pl.BlockSpec((PR, DPAD), lambda i: (i, 0)),
        out_shape=jax.ShapeDtypeStruct((VOCAB, DPAD), jnp.float32),
    )(emb_table)


def _sc_gather(x_r, table_p):
    mesh = plsc.VectorSubcoreMesh(core_axis_name="c", subcore_axis_name="s",
                                  num_cores=NC, num_subcores=NS)

    @functools.partial(
        pl.kernel,
        out_type=jax.ShapeDtypeStruct((ROWS_TOTAL, DPAD), jnp.float32),
        mesh=mesh,
        compiler_params=pltpu.CompilerParams(use_tc_tiling_on_sc=True),
        scratch_types=[
            pltpu.VMEM((NCHUNK, G), jnp.int32),
            pltpu.VMEM((G, DPAD), jnp.float32),
            pltpu.VMEM((G, DPAD), jnp.float32),
            pltpu.SemaphoreType.DMA,
            pltpu.SemaphoreType.DMA,
        ],
    )
    def body(x_hbm, table_hbm, e_hbm, idx_v, buf0, buf1, sem0, sem1):
        c = lax.axis_index("c")
        s = lax.axis_index("s")
        pltpu.sync_copy(x_hbm.at[c, s], idx_v)
        base = (c * NS + s) * (B_PER_W * CTX)

        bufs = (buf0, buf1)
        sems = (sem0, sem1)
        copies = [None, None]
        copies[0] = pltpu.async_copy(table_hbm.at[idx_v.at[0]], bufs[0], sems[0])
        for j in range(NCHUNK):
            if j + 1 < NCHUNK:
                copies[(j + 1) % 2] = pltpu.async_copy(
                    table_hbm.at[idx_v.at[j + 1]], bufs[(j + 1) % 2],
                    sems[(j + 1) % 2])
            copies[j % 2].wait()
            pltpu.sync_copy(bufs[j % 2], e_hbm.at[pl.ds(base + j * G, G)])

    return body(x_r, table_p)


BBLK = 64


def _sum_body(e_ref, s_ref):
    for i in range(BBLK):
        s_ref[i, :] = jnp.sum(e_ref[pl.ds(i * CTX, CTX), :], axis=0)


def _ctx_sum(e):
    return pl.pallas_call(
        _sum_body,
        grid=(BATCH // BBLK,),
        in_specs=[pl.BlockSpec((BBLK * CTX, DPAD), lambda i: (i, 0))],
        out_specs=pl.BlockSpec((BBLK, DPAD), lambda i: (i, 0)),
        out_shape=jax.ShapeDtypeStruct((BATCH, DPAD), jnp.float32),
    )(e)


TB = 2048


def _wt_body(w_ref, o_ref):
    o_ref[...] = w_ref[...].T


def _w_transpose(W):
    return pl.pallas_call(
        _wt_body,
        grid=(pl.cdiv(VOCAB, TB),),
        in_specs=[pl.BlockSpec((TB, EMBED), lambda i: (i, 0))],
        out_specs=pl.BlockSpec((EMBED, TB), lambda i: (0, i)),
        out_shape=jax.ShapeDtypeStruct((EMBED, VOCAB), jnp.float32),
    )(W)


TV = 2048
NBUF = 4
NSTEP = VOCAB // TV
TAIL0 = NSTEP * TV
TAIL = VOCAB - TAIL0


def _mm_body(s_ref, w_ref, b_ref, out_hbm, *scratch):
    bufs = scratch[:NBUF]
    sems = scratch[NBUF:]
    i = pl.program_id(0)
    o = lax.dot_general(
        s_ref[...], w_ref[...], (((1,), (0,)), ((), ())),
        preferred_element_type=jnp.float32) + b_ref[...]
    for kk in range(NBUF):
        @pl.when(i % NBUF == kk)
        def _(kk=kk):
            @pl.when(i >= NBUF)
            def _():
                pltpu.make_async_copy(
                    bufs[kk],
                    out_hbm.at[:, pl.ds((i - NBUF) * TV, TV)],
                    sems[kk]).wait()
            bufs[kk][...] = o
            pltpu.make_async_copy(
                bufs[kk], out_hbm.at[:, pl.ds(i * TV, TV)],
                sems[kk]).start()
    @pl.when(i == NSTEP - 1)
    def _():
        for st in range(NSTEP - NBUF, NSTEP):
            pltpu.make_async_copy(
                bufs[st % NBUF], out_hbm.at[:, pl.ds(st * TV, TV)],
                sems[st % NBUF]).wait()


def _projection_main(s, Wt, b2d):
    return pl.pallas_call(
        _mm_body,
        grid=(NSTEP,),
        in_specs=[
            pl.BlockSpec((BATCH, EMBED), lambda i: (0, 0)),
            pl.BlockSpec((EMBED, TV), lambda i: (0, i)),
            pl.BlockSpec((1, TV), lambda i: (0, i)),
        ],
        out_specs=pl.BlockSpec(memory_space=pl.ANY),
        out_shape=jax.ShapeDtypeStruct((BATCH, VOCAB), jnp.float32),
        scratch_shapes=(
            [pltpu.VMEM((BATCH, TV), jnp.float32) for _ in range(NBUF)]
            + [pltpu.SemaphoreType.DMA for _ in range(NBUF)]),
    )(s, Wt, b2d)


def _tail_body(s_ref, w_ref, b_ref, o_ref):
    o_ref[...] = lax.dot_general(
        s_ref[...], w_ref[...], (((1,), (1,)), ((), ())),
        preferred_element_type=jnp.float32) + b_ref[...]


def _projection_tail(s, W_tail, b_tail):
    return pl.pallas_call(
        _tail_body,
        out_shape=jax.ShapeDtypeStruct((BATCH, TAIL), jnp.float32),
    )(s, W_tail, b_tail)


def kernel(x, emb_table, W, b):
    table_p = _pad_table(emb_table)
    x_r = x.astype(jnp.int32).reshape(NC, NS, NCHUNK, G)
    e = _sc_gather(x_r, table_p)
    s = _ctx_sum(e)[:, :EMBED]
    Wt = _w_transpose(W)
    b2d = b.reshape(1, VOCAB)
    out = _projection_main(s, Wt, b2d)
    tail = _projection_tail(s, W[TAIL0:], b2d[:, TAIL0:])
    return lax.dynamic_update_slice(out, tail, (0, TAIL0))


# speedup vs baseline: 1.0347x; 1.0347x over previous
import functools

import jax
import jax.numpy as jnp
from jax import lax
from jax.experimental import pallas as pl
from jax.experimental.pallas import tpu as pltpu
from jax.experimental.pallas import tpu_sc as plsc

VOCAB = 100000
EMBED = 200
DPAD = 256
BATCH = 1024
CTX = 50

NC = 2
NS = 16
NW = NC * NS

B_PER_W = BATCH // NW
G = 80
NCHUNK = (B_PER_W * CTX) // G
ROWS_TOTAL = BATCH * CTX

PR = 4000


def _pad_body(t_ref, o_ref):
    o_ref[...] = jnp.concatenate(
        [t_ref[...], jnp.zeros((PR, DPAD - EMBED), jnp.float32)], axis=1)


def _pad_table(emb_table):
    return pl.pallas_call(
        _pad_body,
        grid=(VOCAB // PR,),
        in_specs=[pl.BlockSpec((PR, EMBED), lambda i: (i, 0))],
        out_specs=pl.BlockSpec((PR, DPAD), lambda i: (i, 0)),
        out_shape=jax.ShapeDtypeStruct((VOCAB, DPAD), jnp.float32),
    )(emb_table)


def _sc_gather(x_r, table_p):
    mesh = plsc.VectorSubcoreMesh(core_axis_name="c", subcore_axis_name="s",
                                  num_cores=NC, num_subcores=NS)

    @functools.partial(
        pl.kernel,
        out_type=jax.ShapeDtypeStruct((ROWS_TOTAL, DPAD), jnp.float32),
        mesh=mesh,
        compiler_params=pltpu.CompilerParams(use_tc_tiling_on_sc=True),
        scratch_types=[
            pltpu.VMEM((NCHUNK, G), jnp.int32),
            pltpu.VMEM((G, DPAD), jnp.float32),
            pltpu.VMEM((G, DPAD), jnp.float32),
            pltpu.SemaphoreType.DMA,
            pltpu.SemaphoreType.DMA,
        ],
    )
    def body(x_hbm, table_hbm, e_hbm, idx_v, buf0, buf1, sem0, sem1):
        c = lax.axis_index("c")
        s = lax.axis_index("s")
        pltpu.sync_copy(x_hbm.at[c, s], idx_v)
        base = (c * NS + s) * (B_PER_W * CTX)

        bufs = (buf0, buf1)
        sems = (sem0, sem1)
        copies = [None, None]
        copies[0] = pltpu.async_copy(table_hbm.at[idx_v.at[0]], bufs[0], sems[0])
        for j in range(NCHUNK):
            if j + 1 < NCHUNK:
                copies[(j + 1) % 2] = pltpu.async_copy(
                    table_hbm.at[idx_v.at[j + 1]], bufs[(j + 1) % 2],
                    sems[(j + 1) % 2])
            copies[j % 2].wait()
            pltpu.sync_copy(bufs[j % 2], e_hbm.at[pl.ds(base + j * G, G)])

    return body(x_r, table_p)


BBLK = 64


def _sum_body(e_ref, s_ref):
    for i in range(BBLK):
        s_ref[i, :] = jnp.sum(e_ref[pl.ds(i * CTX, CTX), :], axis=0)


def _ctx_sum(e):
    return pl.pallas_call(
        _sum_body,
        grid=(BATCH // BBLK,),
        in_specs=[pl.BlockSpec((BBLK * CTX, DPAD), lambda i: (i, 0))],
        out_specs=pl.BlockSpec((BBLK, DPAD), lambda i: (i, 0)),
        out_shape=jax.ShapeDtypeStruct((BATCH, DPAD), jnp.float32),
    )(e)


TB = 2048


def _wt_body(w_ref, o_ref):
    o_ref[...] = w_ref[...].T.astype(jnp.bfloat16)


def _w_transpose(W):
    return pl.pallas_call(
        _wt_body,
        grid=(pl.cdiv(VOCAB, TB),),
        in_specs=[pl.BlockSpec((TB, EMBED), lambda i: (i, 0))],
        out_specs=pl.BlockSpec((EMBED, TB), lambda i: (0, i)),
        out_shape=jax.ShapeDtypeStruct((EMBED, VOCAB), jnp.bfloat16),
    )(W)


TV = 2048
NBUF = 4
NSTEP = VOCAB // TV
TAIL0 = NSTEP * TV
TAIL = VOCAB - TAIL0


def _mm_body(s_ref, w_ref, b_ref, out_hbm, *scratch):
    bufs = scratch[:NBUF]
    sems = scratch[NBUF:]
    i = pl.program_id(0)
    o = lax.dot_general(
        s_ref[...].astype(jnp.bfloat16), w_ref[...],
        (((1,), (0,)), ((), ())),
        preferred_element_type=jnp.float32) + b_ref[...]
    for kk in range(NBUF):
        @pl.when(i % NBUF == kk)
        def _(kk=kk):
            @pl.when(i >= NBUF)
            def _():
                pltpu.make_async_copy(
                    bufs[kk],
                    out_hbm.at[:, pl.ds((i - NBUF) * TV, TV)],
                    sems[kk]).wait()
            bufs[kk][...] = o
            pltpu.make_async_copy(
                bufs[kk], out_hbm.at[:, pl.ds(i * TV, TV)],
                sems[kk]).start()
    @pl.when(i == NSTEP - 1)
    def _():
        for st in range(NSTEP - NBUF, NSTEP):
            pltpu.make_async_copy(
                bufs[st % NBUF], out_hbm.at[:, pl.ds(st * TV, TV)],
                sems[st % NBUF]).wait()


def _projection_main(s, Wt, b2d):
    return pl.pallas_call(
        _mm_body,
        grid=(NSTEP,),
        in_specs=[
            pl.BlockSpec((BATCH, EMBED), lambda i: (0, 0)),
            pl.BlockSpec((EMBED, TV), lambda i: (0, i)),
            pl.BlockSpec((1, TV), lambda i: (0, i)),
        ],
        out_specs=pl.BlockSpec(memory_space=pl.ANY),
        out_shape=jax.ShapeDtypeStruct((BATCH, VOCAB), jnp.float32),
        scratch_shapes=(
            [pltpu.VMEM((BATCH, TV), jnp.float32) for _ in range(NBUF)]
            + [pltpu.SemaphoreType.DMA for _ in range(NBUF)]),
    )(s, Wt, b2d)


def _tail_body(s_ref, w_ref, b_ref, o_ref):
    o_ref[...] = lax.dot_general(
        s_ref[...], w_ref[...], (((1,), (1,)), ((), ())),
        preferred_element_type=jnp.float32) + b_ref[...]


def _projection_tail(s, W_tail, b_tail):
    return pl.pallas_call(
        _tail_body,
        out_shape=jax.ShapeDtypeStruct((BATCH, TAIL), jnp.float32),
    )(s, W_tail, b_tail)


def kernel(x, emb_table, W, b):
    table_p = _pad_table(emb_table)
    x_r = x.astype(jnp.int32).reshape(NC, NS, NCHUNK, G)
    e = _sc_gather(x_r, table_p)
    s = _ctx_sum(e)[:, :EMBED]
    Wt = _w_transpose(W)
    b2d = b.reshape(1, VOCAB)
    out = _projection_main(s, Wt, b2d)
    tail = _projection_tail(s, W[TAIL0:], b2d[:, TAIL0:])
    return lax.dynamic_update_slice(out, tail, (0, TAIL0))


# speedup vs baseline: 1.0367x; 1.0019x over previous
import functools

import jax
import jax.numpy as jnp
from jax import lax
from jax.experimental import pallas as pl
from jax.experimental.pallas import tpu as pltpu
from jax.experimental.pallas import tpu_sc as plsc

VOCAB = 100000
EMBED = 200
DPAD = 256
BATCH = 1024
CTX = 50

NC = 2
NS = 16
NW = NC * NS

B_PER_W = BATCH // NW
G = 80
NCHUNK = (B_PER_W * CTX) // G
ROWS_TOTAL = BATCH * CTX

PR = 4000


def _pad_body(t_ref, o_ref):
    o_ref[...] = jnp.concatenate(
        [t_ref[...], jnp.zeros((PR, DPAD - EMBED), jnp.float32)], axis=1)


def _pad_table(emb_table):
    return pl.pallas_call(
        _pad_body,
        grid=(VOCAB // PR,),
        in_specs=[pl.BlockSpec((PR, EMBED), lambda i: (i, 0))],
        out_specs=pl.BlockSpec((PR, DPAD), lambda i: (i, 0)),
        out_shape=jax.ShapeDtypeStruct((VOCAB, DPAD), jnp.float32),
    )(emb_table)


def _sc_gather(x_r, table_p):
    mesh = plsc.VectorSubcoreMesh(core_axis_name="c", subcore_axis_name="s",
                                  num_cores=NC, num_subcores=NS)

    @functools.partial(
        pl.kernel,
        out_type=jax.ShapeDtypeStruct((ROWS_TOTAL, DPAD), jnp.float32),
        mesh=mesh,
        compiler_params=pltpu.CompilerParams(use_tc_tiling_on_sc=True),
        scratch_types=[
            pltpu.VMEM((NCHUNK, G), jnp.int32),
            pltpu.VMEM((G, DPAD), jnp.float32),
            pltpu.VMEM((G, DPAD), jnp.float32),
            pltpu.SemaphoreType.DMA,
            pltpu.SemaphoreType.DMA,
        ],
    )
    def body(x_hbm, table_hbm, e_hbm, idx_v, buf0, buf1, sem0, sem1):
        c = lax.axis_index("c")
        s = lax.axis_index("s")
        pltpu.sync_copy(x_hbm.at[c, s], idx_v)
        base = (c * NS + s) * (B_PER_W * CTX)

        bufs = (buf0, buf1)
        sems = (sem0, sem1)
        copies = [None, None]
        copies[0] = pltpu.async_copy(table_hbm.at[idx_v.at[0]], bufs[0], sems[0])
        for j in range(NCHUNK):
            if j + 1 < NCHUNK:
                copies[(j + 1) % 2] = pltpu.async_copy(
                    table_hbm.at[idx_v.at[j + 1]], bufs[(j + 1) % 2],
                    sems[(j + 1) % 2])
            copies[j % 2].wait()
            pltpu.sync_copy(bufs[j % 2], e_hbm.at[pl.ds(base + j * G, G)])

    return body(x_r, table_p)


BBLK = 64


def _sum_body(e_ref, s_ref):
    for i in range(BBLK):
        s_ref[i, :] = jnp.sum(e_ref[pl.ds(i * CTX, CTX), :], axis=0)


def _ctx_sum(e):
    return pl.pallas_call(
        _sum_body,
        grid=(BATCH // BBLK,),
        in_specs=[pl.BlockSpec((BBLK * CTX, DPAD), lambda i: (i, 0))],
        out_specs=pl.BlockSpec((BBLK, DPAD), lambda i: (i, 0)),
        out_shape=jax.ShapeDtypeStruct((BATCH, DPAD), jnp.float32),
    )(e)


TB = 2048


def _wt_body(w_ref, o_ref):
    o_ref[...] = w_ref[...].T.astype(jnp.bfloat16)


def _w_transpose(W):
    return pl.pallas_call(
        _wt_body,
        grid=(pl.cdiv(VOCAB, TB),),
        in_specs=[pl.BlockSpec((TB, EMBED), lambda i: (i, 0))],
        out_specs=pl.BlockSpec((EMBED, TB), lambda i: (0, i)),
        out_shape=jax.ShapeDtypeStruct((EMBED, VOCAB), jnp.bfloat16),
    )(W)


TV = 2048
NBUF = 6
NSTEP = VOCAB // TV
TAIL0 = NSTEP * TV
TAIL = VOCAB - TAIL0


def _mm_body(s_ref, w_ref, b_ref, out_hbm, *scratch):
    bufs = scratch[:NBUF]
    sems = scratch[NBUF:]
    i = pl.program_id(0)
    o = lax.dot_general(
        s_ref[...].astype(jnp.bfloat16), w_ref[...],
        (((1,), (0,)), ((), ())),
        preferred_element_type=jnp.float32) + b_ref[...]
    for kk in range(NBUF):
        @pl.when(i % NBUF == kk)
        def _(kk=kk):
            @pl.when(i >= NBUF)
            def _():
                pltpu.make_async_copy(
                    bufs[kk],
                    out_hbm.at[:, pl.ds((i - NBUF) * TV, TV)],
                    sems[kk]).wait()
            bufs[kk][...] = o
            pltpu.make_async_copy(
                bufs[kk], out_hbm.at[:, pl.ds(i * TV, TV)],
                sems[kk]).start()
    @pl.when(i == NSTEP - 1)
    def _():
        for st in range(NSTEP - NBUF, NSTEP):
            pltpu.make_async_copy(
                bufs[st % NBUF], out_hbm.at[:, pl.ds(st * TV, TV)],
                sems[st % NBUF]).wait()


def _projection_main(s, Wt, b2d):
    return pl.pallas_call(
        _mm_body,
        grid=(NSTEP,),
        in_specs=[
            pl.BlockSpec((BATCH, EMBED), lambda i: (0, 0)),
            pl.BlockSpec((EMBED, TV), lambda i: (0, i)),
            pl.BlockSpec((1, TV), lambda i: (0, i)),
        ],
        out_specs=pl.BlockSpec(memory_space=pl.ANY),
        out_shape=jax.ShapeDtypeStruct((BATCH, VOCAB), jnp.float32),
        scratch_shapes=(
            [pltpu.VMEM((BATCH, TV), jnp.float32) for _ in range(NBUF)]
            + [pltpu.SemaphoreType.DMA for _ in range(NBUF)]),
    )(s, Wt, b2d)


def _tail_body(s_ref, w_ref, b_ref, o_ref):
    o_ref[...] = lax.dot_general(
        s_ref[...], w_ref[...], (((1,), (1,)), ((), ())),
        preferred_element_type=jnp.float32) + b_ref[...]


def _projection_tail(s, W_tail, b_tail):
    return pl.pallas_call(
        _tail_body,
        out_shape=jax.ShapeDtypeStruct((BATCH, TAIL), jnp.float32),
    )(s, W_tail, b_tail)


def kernel(x, emb_table, W, b):
    table_p = _pad_table(emb_table)
    x_r = x.astype(jnp.int32).reshape(NC, NS, NCHUNK, G)
    e = _sc_gather(x_r, table_p)
    s = _ctx_sum(e)[:, :EMBED]
    Wt = _w_transpose(W)
    b2d = b.reshape(1, VOCAB)
    out = _projection_main(s, Wt, b2d)
    tail = _projection_tail(s, W[TAIL0:], b2d[:, TAIL0:])
    return lax.dynamic_update_slice(out, tail, (0, TAIL0))
